# baseline (speedup 1.0000x reference)
import functools

import jax
import jax.numpy as jnp
from jax import lax
from jax.experimental import pallas as pl
from jax.experimental.pallas import tpu as pltpu
from jax.experimental.pallas import tpu_sc as plsc

_BATCH = 4
_SEQ = 8192
_D = 1024
_NC = 2
_NS = 16
_NW = _NC * _NS
_S_PER_W = _SEQ // _NW
_CHUNK = 8
_NCHUNK = _S_PER_W // _CHUNK


def _make_sc_kernel():
    mesh = plsc.VectorSubcoreMesh(core_axis_name="c", subcore_axis_name="s")

    @functools.partial(
        pl.kernel,
        mesh=mesh,
        out_type=jax.ShapeDtypeStruct((_BATCH * _SEQ, _D), jnp.float32),
        scratch_types=[
            pltpu.VMEM((2, _CHUNK, _D), jnp.float32),
            pltpu.VMEM((2, _BATCH, _CHUNK, _D), jnp.float32),
            pltpu.SemaphoreType.DMA((2,)),
            pltpu.SemaphoreType.DMA((2, _BATCH)),
            pltpu.SemaphoreType.DMA((2, _BATCH)),
        ],
    )
    def k(x_hbm, pos_hbm, out_hbm, pbufs, xbufs, psem, xsem, ssem):
        wid = lax.axis_index("s") * _NC + lax.axis_index("c")
        base = wid * _S_PER_W

        def pos_copy(ci):
            pb = lax.rem(ci, 2)
            return pltpu.make_async_copy(
                pos_hbm.at[pl.ds(base + ci * _CHUNK, _CHUNK)],
                pbufs.at[pb],
                psem.at[pb],
            )

        def x_copy(ci, b):
            par = lax.rem(ci, 2)
            row0 = b * _SEQ + base + ci * _CHUNK
            return pltpu.make_async_copy(
                x_hbm.at[pl.ds(row0, _CHUNK)],
                xbufs.at[par, b],
                xsem.at[par, b],
            )

        def store_copy(ci, b):
            par = lax.rem(ci, 2)
            row0 = b * _SEQ + base + ci * _CHUNK
            return pltpu.make_async_copy(
                xbufs.at[par, b],
                out_hbm.at[pl.ds(row0, _CHUNK)],
                ssem.at[par, b],
            )

        pos_copy(0).start()
        for b in range(_BATCH):
            x_copy(0, b).start()

        @pl.loop(0, _NCHUNK)
        def _(ci):
            par = lax.rem(ci, 2)

            @pl.when(ci + 1 < _NCHUNK)
            def _():
                @pl.when(ci >= 1)
                def _():
                    for b in range(_BATCH):
                        store_copy(ci - 1, b).wait()

                for b in range(_BATCH):
                    x_copy(ci + 1, b).start()

            @pl.when(ci + 1 < _NCHUNK)
            def _():
                pos_copy(ci + 1).start()

            pos_copy(ci).wait()
            for b in range(_BATCH):
                x_copy(ci, b).wait()

            @plsc.parallel_loop(0, _CHUNK, unroll=2)
            def _(r):
                for c in range(_D // 16):
                    sl = pl.ds(c * 16, 16)
                    pb = lax.rem(ci, 2)
                    v = pbufs[pb, r, sl]
                    for b in range(_BATCH):
                        plsc.addupdate(xbufs.at[par, b, r, sl], v)

            for b in range(_BATCH):
                store_copy(ci, b).start()

        for ci in (_NCHUNK - 2, _NCHUNK - 1):
            for b in range(_BATCH):
                store_copy(ci, b).wait()

    return k


_sc_kernel = _make_sc_kernel()


def kernel(x, pos_table):
    batch, seq_len, d_model = x.shape
    xf = x.reshape(batch * seq_len, d_model)
    out = _sc_kernel(xf, pos_table[:seq_len])
    return out.reshape(batch, seq_len, d_model)


# speedup vs baseline: 1.1130x; 1.1130x over previous
import functools

import jax
import jax.numpy as jnp
from jax import lax
from jax.experimental import pallas as pl
from jax.experimental.pallas import tpu as pltpu
from jax.experimental.pallas import tpu_sc as plsc

_BATCH = 4
_SEQ = 8192
_D = 1024
_NC = 2
_NS = 16
_NW = _NC * _NS
_S_PER_W = _SEQ // _NW
_CHUNK = 16
_NCHUNK = _S_PER_W // _CHUNK
_NSTEP = _NCHUNK * _BATCH
_NBUF = 6
_PREF = 4


def _make_sc_kernel():
    mesh = plsc.VectorSubcoreMesh(core_axis_name="c", subcore_axis_name="s")

    @functools.partial(
        pl.kernel,
        mesh=mesh,
        out_type=jax.ShapeDtypeStruct((_BATCH * _SEQ, _D), jnp.float32),
        scratch_types=[
            pltpu.VMEM((_CHUNK, _D), jnp.float32),
            pltpu.VMEM((_NBUF, _CHUNK, _D), jnp.float32),
            pltpu.SemaphoreType.DMA,
            pltpu.SemaphoreType.DMA((_NBUF,)),
            pltpu.SemaphoreType.DMA((_NBUF,)),
        ],
    )
    def k(x_hbm, pos_hbm, out_hbm, pbuf, xbufs, psem, xsem, ssem):
        wid = lax.axis_index("s") * _NC + lax.axis_index("c")
        base = wid * _S_PER_W

        def pos_copy(ci):
            return pltpu.make_async_copy(
                pos_hbm.at[pl.ds(base + ci * _CHUNK, _CHUNK)], pbuf, psem
            )

        def x_copy(s):
            ci = lax.div(s, _BATCH)
            b = lax.rem(s, _BATCH)
            xb = lax.rem(s, _NBUF)
            row0 = b * _SEQ + base + ci * _CHUNK
            return pltpu.make_async_copy(
                x_hbm.at[pl.ds(row0, _CHUNK)], xbufs.at[xb], xsem.at[xb]
            )

        def store_copy(s):
            ci = lax.div(s, _BATCH)
            b = lax.rem(s, _BATCH)
            xb = lax.rem(s, _NBUF)
            row0 = b * _SEQ + base + ci * _CHUNK
            return pltpu.make_async_copy(
                xbufs.at[xb], out_hbm.at[pl.ds(row0, _CHUNK)], ssem.at[xb]
            )

        pos_copy(0).start()
        for s in range(_PREF):
            x_copy(s).start()

        @pl.loop(0, _NSTEP)
        def _(s):
            ci = lax.div(s, _BATCH)
            b = lax.rem(s, _BATCH)
            xb = lax.rem(s, _NBUF)

            @pl.when(s + _PREF < _NSTEP)
            def _():
                @pl.when(s >= 2)
                def _():
                    store_copy(s - 2).wait()

                x_copy(s + _PREF).start()

            @pl.when(b == 0)
            def _():
                pos_copy(ci).wait()

            x_copy(s).wait()

            @plsc.parallel_loop(0, _CHUNK, unroll=4)
            def _(r):
                for c in range(_D // 16):
                    v = pbuf[r, pl.ds(c * 16, 16)]
                    plsc.addupdate(xbufs.at[xb, r, pl.ds(c * 16, 16)], v)

            store_copy(s).start()

            @pl.when((b == _BATCH - 1) & (ci + 1 < _NCHUNK))
            def _():
                pos_copy(ci + 1).start()

        for s in range(_NSTEP - _NBUF, _NSTEP):
            store_copy(s).wait()

    return k


_sc_kernel = _make_sc_kernel()


def kernel(x, pos_table):
    batch, seq_len, d_model = x.shape
    xf = x.reshape(batch * seq_len, d_model)
    out = _sc_kernel(xf, pos_table[:seq_len])
    return out.reshape(batch, seq_len, d_model)
